# baseline (speedup 1.0000x reference)
import jax
import jax.numpy as jnp
from jax import lax
from jax.experimental import pallas as pl
from jax.experimental.pallas import tpu as pltpu
from jax.experimental.pallas import tpu_sc as plsc

_N = 4096
_R_SC = 512
_R_TC = _N - _R_SC
_TC_BLK = 256

_NC = 2
_NS = 16
_NW = _NC * _NS
_ROWS_W = _R_SC // _NW
_CR = 4
_NCHUNK = _ROWS_W // _CR
_NBUF = 2
_NPHI = (_N + 1) // 3
_FULLV = _NPHI // 16
_TAIL = _NPHI - _FULLV * 16


def _sc_body(shift_hbm, in_hbm, out_hbm, shift_v, *bufs_and_sems):
    bufs = bufs_and_sems[:_NBUF]
    sem_in, sem_out = bufs_and_sems[_NBUF], bufs_and_sems[_NBUF + 1]
    wid = lax.axis_index("s") * _NC + lax.axis_index("c")
    row0 = _R_TC + wid * _ROWS_W

    pltpu.sync_copy(shift_hbm, shift_v)
    shift = shift_v[...]
    lane = lax.iota(jnp.int32, 16)
    col0 = 1 + 3 * lane
    tail_mask = lane < _TAIL

    def in_slice(kk):
        return in_hbm.at[pl.ds(row0 + kk * _CR, _CR), :]

    def out_slice(kk):
        return out_hbm.at[pl.ds(wid * _ROWS_W + kk * _CR, _CR), :]

    def start_in(kk, b):
        pltpu.async_copy(in_slice(kk), bufs[b], sem_in.at[b])

    def wait_in(kk, b):
        pltpu.make_async_copy(in_slice(kk), bufs[b], sem_in.at[b]).wait()

    def start_out(kk, b):
        pltpu.async_copy(bufs[b], out_slice(kk), sem_out.at[b])

    def wait_out(kk, b):
        pltpu.make_async_copy(bufs[b], out_slice(kk), sem_out.at[b]).wait()

    def transform(x):
        t = (x + shift) - 1.0
        return jnp.where(t > 1.0, t - 2.0, t)

    def process(buf):
        for r in range(_CR):
            rowv = jnp.full((16,), r, jnp.int32)

            @pl.loop(0, _FULLV, init_carry=col0)
            def _(v, col):
                x = plsc.load_gather(buf, [rowv, col])
                plsc.store_scatter(buf, [rowv, col], transform(x))
                return col + 48

            colt = col0 + 48 * _FULLV
            x = plsc.load_gather(buf, [rowv, colt], mask=tail_mask)
            plsc.store_scatter(buf, [rowv, colt], transform(x), mask=tail_mask)

    for b in range(_NBUF):
        start_in(b, b)

    for kk in range(_NBUF):
        if kk + 1 >= _NBUF and kk + 1 < _NCHUNK:
            wait_out(kk + 1 - _NBUF, (kk + 1) % _NBUF)
            start_in(kk + 1, (kk + 1) % _NBUF)
        wait_in(kk, kk % _NBUF)
        process(bufs[kk % _NBUF])
        start_out(kk, kk % _NBUF)

    def steady(g, _):
        for b in range(_NBUF):
            kk = g * _NBUF + b
            b_next = (b + 1) % _NBUF
            wait_out(kk + 1 - _NBUF, b_next)
            start_in(kk + 1, b_next)
            wait_in(kk, b)
            process(bufs[b])
            start_out(kk, b)
        return 0

    if (_NCHUNK - 1) // _NBUF > 1:
        lax.fori_loop(1, (_NCHUNK - 1) // _NBUF, steady, 0)

    for kk in range(max(_NBUF, _NBUF * ((_NCHUNK - 1) // _NBUF)), _NCHUNK):
        if _NBUF <= kk + 1 < _NCHUNK:
            wait_out(kk + 1 - _NBUF, (kk + 1) % _NBUF)
            start_in(kk + 1, (kk + 1) % _NBUF)
        wait_in(kk, kk % _NBUF)
        process(bufs[kk % _NBUF])
        start_out(kk, kk % _NBUF)

    for kk in range(max(_NCHUNK - _NBUF, 0), _NCHUNK):
        wait_out(kk, kk % _NBUF)


_TC_SPLIT = _R_TC // _TC_BLK


def _tc_block_kernel(shift_ref, x_ref, o_ref):
    x = x_ref[...]
    shift = shift_ref[0, 0]
    col = jax.lax.broadcasted_iota(jnp.int32, x.shape, 1)
    mask = (col % 3) == 1
    t = (x + shift) - 1.0
    t = jnp.where(t > 1.0, t - 2.0, t)
    o_ref[...] = jnp.where(mask, t, x)


def _merge_kernel(sc_ref, full_ref, o_ref):
    o_ref[...] = sc_ref[...]


def kernel(input, noise):
    shift11 = (noise * 2.0).reshape(1, 1)
    tc_full = pl.pallas_call(
        _tc_block_kernel,
        grid=(_TC_SPLIT,),
        in_specs=[
            pl.BlockSpec(memory_space=pltpu.SMEM),
            pl.BlockSpec((_TC_BLK, _N), lambda i: (i, 0)),
        ],
        out_specs=pl.BlockSpec((_TC_BLK, _N), lambda i: (i, 0)),
        out_shape=jax.ShapeDtypeStruct((_N, _N), jnp.float32),
        compiler_params=pltpu.CompilerParams(
            dimension_semantics=("parallel",),
        ),
    )(shift11, input)

    shift16 = jnp.broadcast_to(noise * 2.0, (16,))
    sc_out = pl.kernel(
        _sc_body,
        out_type=jax.ShapeDtypeStruct((_R_SC, _N), jnp.float32),
        mesh=plsc.VectorSubcoreMesh(
            core_axis_name="c", subcore_axis_name="s",
            num_cores=_NC, num_subcores=_NS,
        ),
        compiler_params=pltpu.CompilerParams(needs_layout_passes=False),
        scratch_types=[pltpu.VMEM((16,), jnp.float32)]
        + [pltpu.VMEM((_CR, _N), jnp.float32) for _ in range(_NBUF)]
        + [pltpu.SemaphoreType.DMA((_NBUF,)), pltpu.SemaphoreType.DMA((_NBUF,))],
    )(shift16, input)

    return pl.pallas_call(
        _merge_kernel,
        grid=(_R_SC // _TC_BLK,),
        in_specs=[
            pl.BlockSpec((_TC_BLK, _N), lambda i: (i, 0)),
            pl.BlockSpec(memory_space=pl.ANY),
        ],
        out_specs=pl.BlockSpec((_TC_BLK, _N), lambda i: (i + _TC_SPLIT, 0)),
        out_shape=jax.ShapeDtypeStruct((_N, _N), jnp.float32),
        input_output_aliases={1: 0},
        compiler_params=pltpu.CompilerParams(
            dimension_semantics=("arbitrary",),
        ),
    )(sc_out, tc_full)


# speedup vs baseline: 1.0327x; 1.0327x over previous
import jax
import jax.numpy as jnp
from jax import lax
from jax.experimental import pallas as pl
from jax.experimental.pallas import tpu as pltpu
from jax.experimental.pallas import tpu_sc as plsc

_N = 4096
_R_SC = 256
_R_TC = _N - _R_SC
_TC_BLK = 256

_NC = 2
_NS = 16
_NW = _NC * _NS
_ROWS_W = _R_SC // _NW
_CR = 4
_NCHUNK = _ROWS_W // _CR
_NBUF = 2
_NPHI = (_N + 1) // 3
_FULLV = _NPHI // 16
_TAIL = _NPHI - _FULLV * 16


def _sc_body(shift_hbm, in_hbm, out_hbm, shift_v, *bufs_and_sems):
    bufs = bufs_and_sems[:_NBUF]
    sem_in, sem_out = bufs_and_sems[_NBUF], bufs_and_sems[_NBUF + 1]
    wid = lax.axis_index("s") * _NC + lax.axis_index("c")
    row0 = _R_TC + wid * _ROWS_W

    pltpu.sync_copy(shift_hbm, shift_v)
    shift = shift_v[...]
    lane = lax.iota(jnp.int32, 16)
    col0 = 1 + 3 * lane
    tail_mask = lane < _TAIL

    def in_slice(kk):
        return in_hbm.at[pl.ds(row0 + kk * _CR, _CR), :]

    def out_slice(kk):
        return out_hbm.at[pl.ds(wid * _ROWS_W + kk * _CR, _CR), :]

    def start_in(kk, b):
        pltpu.async_copy(in_slice(kk), bufs[b], sem_in.at[b])

    def wait_in(kk, b):
        pltpu.make_async_copy(in_slice(kk), bufs[b], sem_in.at[b]).wait()

    def start_out(kk, b):
        pltpu.async_copy(bufs[b], out_slice(kk), sem_out.at[b])

    def wait_out(kk, b):
        pltpu.make_async_copy(bufs[b], out_slice(kk), sem_out.at[b]).wait()

    def transform(x):
        t = (x + shift) - 1.0
        return jnp.where(t > 1.0, t - 2.0, t)

    def process(buf):
        for r in range(_CR):
            rowv = jnp.full((16,), r, jnp.int32)

            @pl.loop(0, _FULLV, init_carry=col0)
            def _(v, col):
                x = plsc.load_gather(buf, [rowv, col])
                plsc.store_scatter(buf, [rowv, col], transform(x))
                return col + 48

            colt = col0 + 48 * _FULLV
            x = plsc.load_gather(buf, [rowv, colt], mask=tail_mask)
            plsc.store_scatter(buf, [rowv, colt], transform(x), mask=tail_mask)

    for b in range(_NBUF):
        start_in(b, b)

    for kk in range(_NBUF):
        if kk + 1 >= _NBUF and kk + 1 < _NCHUNK:
            wait_out(kk + 1 - _NBUF, (kk + 1) % _NBUF)
            start_in(kk + 1, (kk + 1) % _NBUF)
        wait_in(kk, kk % _NBUF)
        process(bufs[kk % _NBUF])
        start_out(kk, kk % _NBUF)

    def steady(g, _):
        for b in range(_NBUF):
            kk = g * _NBUF + b
            b_next = (b + 1) % _NBUF
            wait_out(kk + 1 - _NBUF, b_next)
            start_in(kk + 1, b_next)
            wait_in(kk, b)
            process(bufs[b])
            start_out(kk, b)
        return 0

    if (_NCHUNK - 1) // _NBUF > 1:
        lax.fori_loop(1, (_NCHUNK - 1) // _NBUF, steady, 0)

    for kk in range(max(_NBUF, _NBUF * ((_NCHUNK - 1) // _NBUF)), _NCHUNK):
        if _NBUF <= kk + 1 < _NCHUNK:
            wait_out(kk + 1 - _NBUF, (kk + 1) % _NBUF)
            start_in(kk + 1, (kk + 1) % _NBUF)
        wait_in(kk, kk % _NBUF)
        process(bufs[kk % _NBUF])
        start_out(kk, kk % _NBUF)

    for kk in range(max(_NCHUNK - _NBUF, 0), _NCHUNK):
        wait_out(kk, kk % _NBUF)


_TC_SPLIT = _R_TC // _TC_BLK


def _tc_block_kernel(shift_ref, x_ref, o_ref):
    x = x_ref[...]
    shift = shift_ref[0, 0]
    col = jax.lax.broadcasted_iota(jnp.int32, x.shape, 1)
    mask = (col % 3) == 1
    t = (x + shift) - 1.0
    t = jnp.where(t > 1.0, t - 2.0, t)
    o_ref[...] = jnp.where(mask, t, x)


def _merge_kernel(sc_ref, full_ref, o_ref):
    o_ref[...] = sc_ref[...]


def kernel(input, noise):
    shift11 = (noise * 2.0).reshape(1, 1)
    tc_full = pl.pallas_call(
        _tc_block_kernel,
        grid=(_TC_SPLIT,),
        in_specs=[
            pl.BlockSpec(memory_space=pltpu.SMEM),
            pl.BlockSpec((_TC_BLK, _N), lambda i: (i, 0)),
        ],
        out_specs=pl.BlockSpec((_TC_BLK, _N), lambda i: (i, 0)),
        out_shape=jax.ShapeDtypeStruct((_N, _N), jnp.float32),
        compiler_params=pltpu.CompilerParams(
            dimension_semantics=("parallel",),
        ),
    )(shift11, input)

    shift16 = jnp.broadcast_to(noise * 2.0, (16,))
    sc_out = pl.kernel(
        _sc_body,
        out_type=jax.ShapeDtypeStruct((_R_SC, _N), jnp.float32),
        mesh=plsc.VectorSubcoreMesh(
            core_axis_name="c", subcore_axis_name="s",
            num_cores=_NC, num_subcores=_NS,
        ),
        compiler_params=pltpu.CompilerParams(needs_layout_passes=False),
        scratch_types=[pltpu.VMEM((16,), jnp.float32)]
        + [pltpu.VMEM((_CR, _N), jnp.float32) for _ in range(_NBUF)]
        + [pltpu.SemaphoreType.DMA((_NBUF,)), pltpu.SemaphoreType.DMA((_NBUF,))],
    )(shift16, input)

    return pl.pallas_call(
        _merge_kernel,
        grid=(_R_SC // _TC_BLK,),
        in_specs=[
            pl.BlockSpec((_TC_BLK, _N), lambda i: (i, 0)),
            pl.BlockSpec(memory_space=pl.ANY),
        ],
        out_specs=pl.BlockSpec((_TC_BLK, _N), lambda i: (i + _TC_SPLIT, 0)),
        out_shape=jax.ShapeDtypeStruct((_N, _N), jnp.float32),
        input_output_aliases={1: 0},
        compiler_params=pltpu.CompilerParams(
            dimension_semantics=("arbitrary",),
        ),
    )(sc_out, tc_full)
